# Initial kernel scaffold; baseline (speedup 1.0000x reference)
import jax
import jax.numpy as jnp
from jax.experimental import pallas as pl


def kernel(x, edge_index, w1, b1, w2, b2):
    raise NotImplementedError("write your pallas kernel here")


# speedup vs baseline: 19.8336x; 19.8336x over previous
import functools

import jax
import jax.numpy as jnp
from jax import lax
from jax.experimental import pallas as pl
from jax.experimental.pallas import tpu as pltpu
from jax.experimental.pallas import tpu_sc as plsc

N = 10000
E = 320000
D_IN = 128
H = 256
C = 40

_INFO = plsc.get_sparse_core_info()
NC, NS = _INFO.num_cores, _INFO.num_subcores

NPAD = 10240
ROWS_PER_TILE = NPAD // NS
EDGES_PER_TILE = E // (NC * NS)
CHUNK = 80
NCHUNKS = EDGES_PER_TILE // CHUNK

_MESH = plsc.VectorSubcoreMesh(core_axis_name="c", subcore_axis_name="s")


@functools.partial(
    pl.kernel,
    out_type=jax.ShapeDtypeStruct((NC, NPAD), jnp.float32),
    mesh=_MESH,
    scratch_types=[
        pltpu.VMEM((CHUNK,), jnp.int32),
        pltpu.VMEM((CHUNK,), jnp.float32),
        pltpu.VMEM_SHARED((NPAD,), jnp.float32),
    ],
)
def _deg_kernel(dst_hbm, zeros_hbm, ones_hbm, deg_out, idx_v, ones_v, deg_sh):
    cid = lax.axis_index("c")
    sid = lax.axis_index("s")
    r0 = sid * ROWS_PER_TILE
    pltpu.sync_copy(zeros_hbm, deg_sh.at[pl.ds(r0, ROWS_PER_TILE)])
    pltpu.sync_copy(ones_hbm, ones_v)
    plsc.subcore_barrier()
    ebase = (cid * NS + sid) * EDGES_PER_TILE

    def body(i, carry):
        pltpu.sync_copy(dst_hbm.at[pl.ds(ebase + i * CHUNK, CHUNK)], idx_v)
        pltpu.sync_copy(ones_v, deg_sh.at[idx_v], add=True)
        return carry

    lax.fori_loop(0, NCHUNKS, body, 0)
    plsc.subcore_barrier()
    pltpu.sync_copy(deg_sh.at[pl.ds(r0, ROWS_PER_TILE)],
                    deg_out.at[cid, pl.ds(r0, ROWS_PER_TILE)])


@functools.partial(
    pl.kernel,
    out_type=jax.ShapeDtypeStruct((NC, NPAD, D_IN), jnp.float32),
    mesh=_MESH,
    scratch_types=[
        pltpu.VMEM((CHUNK,), jnp.int32),
        pltpu.VMEM((CHUNK,), jnp.int32),
        pltpu.VMEM((CHUNK, D_IN), jnp.float32),
        pltpu.VMEM_SHARED((NPAD, D_IN), jnp.float32),
        pltpu.SemaphoreType.DMA,
    ],
)
def _agg_kernel(u_hbm, src_hbm, dst_hbm, zrows_hbm, z_out,
                sidx_v, didx_v, rows_v, z_sh, sem):
    cid = lax.axis_index("c")
    sid = lax.axis_index("s")
    r0 = sid * ROWS_PER_TILE
    pltpu.sync_copy(zrows_hbm, z_sh.at[pl.ds(r0, ROWS_PER_TILE)])
    plsc.subcore_barrier()
    ebase = (cid * NS + sid) * EDGES_PER_TILE

    def body(i, carry):
        e0 = ebase + i * CHUNK
        pltpu.sync_copy(src_hbm.at[pl.ds(e0, CHUNK)], sidx_v)
        pltpu.sync_copy(dst_hbm.at[pl.ds(e0, CHUNK)], didx_v)
        pltpu.async_copy(u_hbm.at[sidx_v], rows_v, sem).wait()
        pltpu.sync_copy(rows_v, z_sh.at[didx_v], add=True)
        return carry

    lax.fori_loop(0, NCHUNKS, body, 0)
    plsc.subcore_barrier()
    pltpu.sync_copy(z_sh.at[pl.ds(r0, ROWS_PER_TILE)],
                    z_out.at[cid, pl.ds(r0, ROWS_PER_TILE)])


_ROWBLK = 1024


def _scale_body(degt_ref, x_ref, u_ref, dinv_ref):
    deg = degt_ref[:, 0:1] + degt_ref[:, 1:2] + 1.0
    dinv = lax.rsqrt(deg)
    dinv_ref[...] = dinv
    u_ref[...] = x_ref[...] * dinv


def _scale(degt, xpad):
    grid = (NPAD // _ROWBLK,)
    return pl.pallas_call(
        _scale_body,
        grid=grid,
        in_specs=[
            pl.BlockSpec((_ROWBLK, NC), lambda i: (i, 0)),
            pl.BlockSpec((_ROWBLK, D_IN), lambda i: (i, 0)),
        ],
        out_specs=[
            pl.BlockSpec((_ROWBLK, D_IN), lambda i: (i, 0)),
            pl.BlockSpec((_ROWBLK, 1), lambda i: (i, 0)),
        ],
        out_shape=[
            jax.ShapeDtypeStruct((NPAD, D_IN), jnp.float32),
            jax.ShapeDtypeStruct((NPAD, 1), jnp.float32),
        ],
    )(degt, xpad)


def _dense_body(z_ref, u_ref, dinv_ref, w1_ref, b1_ref, w2_ref, b2_ref,
                out_ref):
    zsum = z_ref[0] + z_ref[1] + u_ref[...]
    y = zsum * dinv_ref[...]
    h = jnp.dot(y, w1_ref[...], preferred_element_type=jnp.float32)
    h = jnp.maximum(h + b1_ref[...], 0.0)
    o = jnp.dot(h, w2_ref[...], preferred_element_type=jnp.float32)
    o = o + b2_ref[...]
    m = jnp.max(o, axis=1, keepdims=True)
    lse = jnp.log(jnp.sum(jnp.exp(o - m), axis=1, keepdims=True)) + m
    out_ref[...] = o - lse


def _dense(zp, u, dinv, w1, b1, w2, b2):
    grid = (NPAD // _ROWBLK,)
    return pl.pallas_call(
        _dense_body,
        grid=grid,
        in_specs=[
            pl.BlockSpec((NC, _ROWBLK, D_IN), lambda i: (0, i, 0)),
            pl.BlockSpec((_ROWBLK, D_IN), lambda i: (i, 0)),
            pl.BlockSpec((_ROWBLK, 1), lambda i: (i, 0)),
            pl.BlockSpec((D_IN, H), lambda i: (0, 0)),
            pl.BlockSpec((1, H), lambda i: (0, 0)),
            pl.BlockSpec((H, C), lambda i: (0, 0)),
            pl.BlockSpec((1, C), lambda i: (0, 0)),
        ],
        out_specs=pl.BlockSpec((_ROWBLK, C), lambda i: (i, 0)),
        out_shape=jax.ShapeDtypeStruct((NPAD, C), jnp.float32),
    )(zp, u, dinv, w1, b1, w2, b2)


def kernel(x, edge_index, w1, b1, w2, b2):
    src = edge_index[0]
    dst = edge_index[1]
    xpad = jnp.zeros((NPAD, D_IN), jnp.float32).at[:N].set(x)
    zeros_deg = jnp.zeros((ROWS_PER_TILE,), jnp.float32)
    ones_chunk = jnp.ones((CHUNK,), jnp.float32)
    zeros_rows = jnp.zeros((ROWS_PER_TILE, D_IN), jnp.float32)

    degp = _deg_kernel(dst, zeros_deg, ones_chunk)
    degt = degp.T
    u, dinv = _scale(degt, xpad)
    zp = _agg_kernel(u, src, dst, zeros_rows)
    out = _dense(zp, u, dinv, w1, b1.reshape(1, H), w2, b2.reshape(1, C))
    return out[:N]


# speedup vs baseline: 50.7135x; 2.5569x over previous
import functools

import jax
import jax.numpy as jnp
from jax import lax
from jax.experimental import pallas as pl
from jax.experimental.pallas import tpu as pltpu
from jax.experimental.pallas import tpu_sc as plsc

N = 10000
E = 320000
D_IN = 128
H = 256
C = 40

_INFO = plsc.get_sparse_core_info()
NC, NS = _INFO.num_cores, _INFO.num_subcores

NPAD = 10240
ROWS_PER_TILE = NPAD // NS
EDGES_PER_TILE = E // (NC * NS)
CHUNK = 40
NCHUNKS = EDGES_PER_TILE // CHUNK
NBUF = 5
NGROUPS = NCHUNKS // NBUF
DEG_CHUNK = 80
DEG_NCHUNKS = EDGES_PER_TILE // DEG_CHUNK
DEG_WIN = 8

_MESH = plsc.VectorSubcoreMesh(core_axis_name="c", subcore_axis_name="s")


@functools.partial(
    pl.kernel,
    out_type=jax.ShapeDtypeStruct((NC, NPAD), jnp.float32),
    mesh=_MESH,
    scratch_types=[
        pltpu.VMEM((DEG_NCHUNKS, DEG_CHUNK), jnp.int32),
        pltpu.VMEM((DEG_CHUNK,), jnp.float32),
        pltpu.VMEM_SHARED((NPAD,), jnp.float32),
        pltpu.SemaphoreType.DMA,
    ],
)
def _deg_kernel(dst3_hbm, zeros_hbm, ones_hbm, deg_out, idx_v, ones_v,
                deg_sh, dsem):
    cid = lax.axis_index("c")
    sid = lax.axis_index("s")
    r0 = sid * ROWS_PER_TILE
    tid = cid * NS + sid
    pltpu.sync_copy(zeros_hbm, deg_sh.at[pl.ds(r0, ROWS_PER_TILE)])
    pltpu.sync_copy(ones_hbm, ones_v)
    pltpu.sync_copy(dst3_hbm.at[tid], idx_v)
    plsc.subcore_barrier()

    for i in range(DEG_WIN):
        pltpu.async_copy(ones_v, deg_sh.at[idx_v.at[i]], dsem, add=True)

    @pl.loop(DEG_WIN, DEG_NCHUNKS)
    def _(i):
        pltpu.make_async_copy(ones_v, deg_sh.at[idx_v.at[0]], dsem).wait()
        pltpu.async_copy(ones_v, deg_sh.at[idx_v.at[i]], dsem, add=True)

    for _ in range(DEG_WIN):
        pltpu.make_async_copy(ones_v, deg_sh.at[idx_v.at[0]], dsem).wait()

    plsc.subcore_barrier()
    pltpu.sync_copy(deg_sh.at[pl.ds(r0, ROWS_PER_TILE)],
                    deg_out.at[cid, pl.ds(r0, ROWS_PER_TILE)])


@functools.partial(
    pl.kernel,
    out_type=jax.ShapeDtypeStruct((NC, NPAD, D_IN), jnp.float32),
    mesh=_MESH,
    scratch_types=[
        pltpu.VMEM((EDGES_PER_TILE,), jnp.int32),
        [pltpu.VMEM((CHUNK,), jnp.int32) for _ in range(NBUF)],
        pltpu.VMEM((NBUF, CHUNK, D_IN), jnp.float32),
        pltpu.VMEM_SHARED((NPAD, D_IN), jnp.float32),
        pltpu.SemaphoreType.DMA((NBUF,)),
        pltpu.SemaphoreType.DMA((NBUF,)),
        pltpu.SemaphoreType.DMA((NBUF,)),
    ],
)
def _agg_kernel(u_hbm, src2_hbm, dstf_hbm, zrows_hbm, z_out,
                sidx, dbufs, rows, z_sh, gsem, ssem, isem):
    cid = lax.axis_index("c")
    sid = lax.axis_index("s")
    r0 = sid * ROWS_PER_TILE
    tid = cid * NS + sid
    ebase = tid * EDGES_PER_TILE
    pltpu.sync_copy(zrows_hbm, z_sh.at[pl.ds(r0, ROWS_PER_TILE)])
    pltpu.sync_copy(src2_hbm.at[tid], sidx)
    plsc.subcore_barrier()

    def fire(i, b):
        pltpu.async_copy(u_hbm.at[sidx.at[pl.ds(i * CHUNK, CHUNK)]],
                         rows.at[b], gsem.at[b])
        pltpu.async_copy(dstf_hbm.at[pl.ds(ebase + i * CHUNK, CHUNK)],
                         dbufs[b], isem.at[b])

    def visit(i, b, fire_next):
        pltpu.make_async_copy(dstf_hbm.at[pl.ds(ebase, CHUNK)],
                              dbufs[b], isem.at[b]).wait()
        pltpu.make_async_copy(u_hbm.at[sidx.at[pl.ds(0, CHUNK)]],
                              rows.at[b], gsem.at[b]).wait()
        pltpu.async_copy(rows.at[b], z_sh.at[dbufs[b]], ssem.at[b],
                         add=True)
        if fire_next:
            pltpu.make_async_copy(rows.at[b], z_sh.at[dbufs[b]],
                                  ssem.at[b]).wait()
            fire(i + NBUF, b)

    for b in range(NBUF):
        fire(b, b)

    @pl.loop(0, NGROUPS - 1)
    def _(g):
        i0 = g * NBUF
        for b in range(NBUF):
            visit(i0 + b, b, True)

    last = (NGROUPS - 1) * NBUF
    for b in range(NBUF):
        visit(last + b, b, False)
    for b in range(NBUF):
        pltpu.make_async_copy(rows.at[b], z_sh.at[dbufs[b]],
                              ssem.at[b]).wait()

    plsc.subcore_barrier()
    pltpu.sync_copy(z_sh.at[pl.ds(r0, ROWS_PER_TILE)],
                    z_out.at[cid, pl.ds(r0, ROWS_PER_TILE)])


_ROWBLK = 1024


def _scale_body(degt_ref, x_ref, u_ref, dinv_ref):
    deg = degt_ref[:, 0:1] + degt_ref[:, 1:2] + 1.0
    dinv = lax.rsqrt(deg)
    dinv_ref[...] = dinv
    u_ref[...] = x_ref[...] * dinv


def _scale(degt, xpad):
    grid = (NPAD // _ROWBLK,)
    return pl.pallas_call(
        _scale_body,
        grid=grid,
        in_specs=[
            pl.BlockSpec((_ROWBLK, NC), lambda i: (i, 0)),
            pl.BlockSpec((_ROWBLK, D_IN), lambda i: (i, 0)),
        ],
        out_specs=[
            pl.BlockSpec((_ROWBLK, D_IN), lambda i: (i, 0)),
            pl.BlockSpec((_ROWBLK, 1), lambda i: (i, 0)),
        ],
        out_shape=[
            jax.ShapeDtypeStruct((NPAD, D_IN), jnp.float32),
            jax.ShapeDtypeStruct((NPAD, 1), jnp.float32),
        ],
    )(degt, xpad)


def _dense_body(z_ref, u_ref, dinv_ref, w1_ref, b1_ref, w2_ref, b2_ref,
                out_ref):
    zsum = z_ref[0] + z_ref[1] + u_ref[...]
    y = zsum * dinv_ref[...]
    h = jnp.dot(y, w1_ref[...], preferred_element_type=jnp.float32)
    h = jnp.maximum(h + b1_ref[...], 0.0)
    o = jnp.dot(h, w2_ref[...], preferred_element_type=jnp.float32)
    o = o + b2_ref[...]
    m = jnp.max(o, axis=1, keepdims=True)
    lse = jnp.log(jnp.sum(jnp.exp(o - m), axis=1, keepdims=True)) + m
    out_ref[...] = o - lse


def _dense(zp, u, dinv, w1, b1, w2, b2):
    grid = (NPAD // _ROWBLK,)
    return pl.pallas_call(
        _dense_body,
        grid=grid,
        in_specs=[
            pl.BlockSpec((NC, _ROWBLK, D_IN), lambda i: (0, i, 0)),
            pl.BlockSpec((_ROWBLK, D_IN), lambda i: (i, 0)),
            pl.BlockSpec((_ROWBLK, 1), lambda i: (i, 0)),
            pl.BlockSpec((D_IN, H), lambda i: (0, 0)),
            pl.BlockSpec((1, H), lambda i: (0, 0)),
            pl.BlockSpec((H, C), lambda i: (0, 0)),
            pl.BlockSpec((1, C), lambda i: (0, 0)),
        ],
        out_specs=pl.BlockSpec((_ROWBLK, C), lambda i: (i, 0)),
        out_shape=jax.ShapeDtypeStruct((NPAD, C), jnp.float32),
    )(zp, u, dinv, w1, b1, w2, b2)


def kernel(x, edge_index, w1, b1, w2, b2):
    src2 = edge_index[0].reshape(NC * NS, EDGES_PER_TILE)
    dstf = edge_index[1]
    dst3d = edge_index[1].reshape(NC * NS, DEG_NCHUNKS, DEG_CHUNK)
    xpad = jnp.zeros((NPAD, D_IN), jnp.float32).at[:N].set(x)
    zeros_deg = jnp.zeros((ROWS_PER_TILE,), jnp.float32)
    ones_chunk = jnp.ones((DEG_CHUNK,), jnp.float32)
    zeros_rows = jnp.zeros((ROWS_PER_TILE, D_IN), jnp.float32)

    degp = _deg_kernel(dst3d, zeros_deg, ones_chunk)
    degt = degp.T
    u, dinv = _scale(degt, xpad)
    zp = _agg_kernel(u, src2, dstf, zeros_rows)
    out = _dense(zp, u, dinv, w1, b1.reshape(1, H), w2, b2.reshape(1, C))
    return out[:N]


# speedup vs baseline: 51.4415x; 1.0144x over previous
import functools

import jax
import jax.numpy as jnp
from jax import lax
from jax.experimental import pallas as pl
from jax.experimental.pallas import tpu as pltpu
from jax.experimental.pallas import tpu_sc as plsc

N = 10000
E = 320000
D_IN = 128
H = 256
C = 40

_INFO = plsc.get_sparse_core_info()
NC, NS = _INFO.num_cores, _INFO.num_subcores

NPAD = 10240
ROWS_PER_TILE = NPAD // NS
EDGES_PER_TILE = E // (NC * NS)
CHUNK = 40
NCHUNKS = EDGES_PER_TILE // CHUNK
NBUF = 5
NGROUPS = NCHUNKS // NBUF
DEG_CHUNK = 80
DEG_NCHUNKS = EDGES_PER_TILE // DEG_CHUNK
DEG_WIN = 8

_MESH = plsc.VectorSubcoreMesh(core_axis_name="c", subcore_axis_name="s")


@functools.partial(
    pl.kernel,
    out_type=jax.ShapeDtypeStruct((NC, NPAD), jnp.float32),
    mesh=_MESH,
    scratch_types=[
        pltpu.VMEM((DEG_NCHUNKS, DEG_CHUNK), jnp.int32),
        pltpu.VMEM((DEG_CHUNK,), jnp.float32),
        pltpu.VMEM_SHARED((NPAD,), jnp.float32),
        pltpu.SemaphoreType.DMA,
    ],
)
def _deg_kernel(dst3_hbm, zeros_hbm, ones_hbm, deg_out, idx_v, ones_v,
                deg_sh, dsem):
    cid = lax.axis_index("c")
    sid = lax.axis_index("s")
    r0 = sid * ROWS_PER_TILE
    tid = cid * NS + sid
    pltpu.sync_copy(zeros_hbm, deg_sh.at[pl.ds(r0, ROWS_PER_TILE)])
    pltpu.sync_copy(ones_hbm, ones_v)
    pltpu.sync_copy(dst3_hbm.at[tid], idx_v)
    plsc.subcore_barrier()

    for i in range(DEG_WIN):
        pltpu.async_copy(ones_v, deg_sh.at[idx_v.at[i]], dsem, add=True)

    @pl.loop(DEG_WIN, DEG_NCHUNKS)
    def _(i):
        pltpu.make_async_copy(ones_v, deg_sh.at[idx_v.at[0]], dsem).wait()
        pltpu.async_copy(ones_v, deg_sh.at[idx_v.at[i]], dsem, add=True)

    for _ in range(DEG_WIN):
        pltpu.make_async_copy(ones_v, deg_sh.at[idx_v.at[0]], dsem).wait()

    plsc.subcore_barrier()
    pltpu.sync_copy(deg_sh.at[pl.ds(r0, ROWS_PER_TILE)],
                    deg_out.at[cid, pl.ds(r0, ROWS_PER_TILE)])


@functools.partial(
    pl.kernel,
    out_type=jax.ShapeDtypeStruct((NC, NPAD, D_IN), jnp.float32),
    mesh=_MESH,
    scratch_types=[
        pltpu.VMEM((EDGES_PER_TILE,), jnp.int32),
        [pltpu.VMEM((CHUNK,), jnp.int32) for _ in range(NBUF)],
        pltpu.VMEM((NBUF, CHUNK, D_IN), jnp.float32),
        pltpu.VMEM_SHARED((NPAD, D_IN), jnp.float32),
        pltpu.SemaphoreType.DMA((NBUF,)),
        pltpu.SemaphoreType.DMA((NBUF,)),
        pltpu.SemaphoreType.DMA((NBUF,)),
    ],
)
def _agg_kernel(u_hbm, src2_hbm, dstf_hbm, zrows_hbm, z_out,
                sidx, dbufs, rows, z_sh, gsem, ssem, isem):
    cid = lax.axis_index("c")
    sid = lax.axis_index("s")
    r0 = sid * ROWS_PER_TILE
    tid = cid * NS + sid
    ebase = tid * EDGES_PER_TILE
    pltpu.sync_copy(zrows_hbm, z_sh.at[pl.ds(r0, ROWS_PER_TILE)])
    pltpu.sync_copy(src2_hbm.at[tid], sidx)
    plsc.subcore_barrier()

    def fire(i, b):
        pltpu.async_copy(u_hbm.at[sidx.at[pl.ds(i * CHUNK, CHUNK)]],
                         rows.at[b], gsem.at[b])
        pltpu.async_copy(dstf_hbm.at[pl.ds(ebase + i * CHUNK, CHUNK)],
                         dbufs[b], isem.at[b])

    def visit(i, b, fire_next):
        pltpu.make_async_copy(dstf_hbm.at[pl.ds(ebase, CHUNK)],
                              dbufs[b], isem.at[b]).wait()
        pltpu.make_async_copy(u_hbm.at[sidx.at[pl.ds(0, CHUNK)]],
                              rows.at[b], gsem.at[b]).wait()
        pltpu.async_copy(rows.at[b], z_sh.at[dbufs[b]], ssem.at[b],
                         add=True)
        if fire_next:
            pltpu.make_async_copy(rows.at[b], z_sh.at[dbufs[b]],
                                  ssem.at[b]).wait()
            fire(i + NBUF, b)

    for b in range(NBUF):
        fire(b, b)

    @pl.loop(0, NGROUPS - 1)
    def _(g):
        i0 = g * NBUF
        for b in range(NBUF):
            visit(i0 + b, b, True)

    last = (NGROUPS - 1) * NBUF
    for b in range(NBUF):
        visit(last + b, b, False)
    for b in range(NBUF):
        pltpu.make_async_copy(rows.at[b], z_sh.at[dbufs[b]],
                              ssem.at[b]).wait()

    plsc.subcore_barrier()
    pltpu.sync_copy(z_sh.at[pl.ds(r0, ROWS_PER_TILE)],
                    z_out.at[cid, pl.ds(r0, ROWS_PER_TILE)])


_ROWBLK = 1000


def _scale_body(degt_ref, x_ref, u_ref, dinv_ref):
    deg = degt_ref[:, 0:1] + degt_ref[:, 1:2] + 1.0
    dinv = lax.rsqrt(deg)
    dinv_ref[...] = dinv
    u_ref[...] = x_ref[...] * dinv


def _scale(degt, x):
    grid = (N // _ROWBLK,)
    return pl.pallas_call(
        _scale_body,
        grid=grid,
        in_specs=[
            pl.BlockSpec((_ROWBLK, NC), lambda i: (i, 0)),
            pl.BlockSpec((_ROWBLK, D_IN), lambda i: (i, 0)),
        ],
        out_specs=[
            pl.BlockSpec((_ROWBLK, D_IN), lambda i: (i, 0)),
            pl.BlockSpec((_ROWBLK, 1), lambda i: (i, 0)),
        ],
        out_shape=[
            jax.ShapeDtypeStruct((N, D_IN), jnp.float32),
            jax.ShapeDtypeStruct((N, 1), jnp.float32),
        ],
    )(degt, x)


def _dense_body(z_ref, u_ref, dinv_ref, w1_ref, b1_ref, w2_ref, b2_ref,
                out_ref):
    zsum = z_ref[0] + z_ref[1] + u_ref[...]
    y = zsum * dinv_ref[...]
    h = jnp.dot(y, w1_ref[...], preferred_element_type=jnp.float32)
    h = jnp.maximum(h + b1_ref[...], 0.0)
    o = jnp.dot(h, w2_ref[...], preferred_element_type=jnp.float32)
    o = o + b2_ref[...]
    m = jnp.max(o, axis=1, keepdims=True)
    lse = jnp.log(jnp.sum(jnp.exp(o - m), axis=1, keepdims=True)) + m
    out_ref[...] = o - lse


def _dense(zp, u, dinv, w1, b1, w2, b2):
    grid = (N // _ROWBLK,)
    return pl.pallas_call(
        _dense_body,
        grid=grid,
        in_specs=[
            pl.BlockSpec((NC, _ROWBLK, D_IN), lambda i: (0, i, 0)),
            pl.BlockSpec((_ROWBLK, D_IN), lambda i: (i, 0)),
            pl.BlockSpec((_ROWBLK, 1), lambda i: (i, 0)),
            pl.BlockSpec((D_IN, H), lambda i: (0, 0)),
            pl.BlockSpec((1, H), lambda i: (0, 0)),
            pl.BlockSpec((H, C), lambda i: (0, 0)),
            pl.BlockSpec((1, C), lambda i: (0, 0)),
        ],
        out_specs=pl.BlockSpec((_ROWBLK, C), lambda i: (i, 0)),
        out_shape=jax.ShapeDtypeStruct((N, C), jnp.float32),
    )(zp, u, dinv, w1, b1, w2, b2)


def kernel(x, edge_index, w1, b1, w2, b2):
    src2 = edge_index[0].reshape(NC * NS, EDGES_PER_TILE)
    dstf = edge_index[1]
    dst3d = edge_index[1].reshape(NC * NS, DEG_NCHUNKS, DEG_CHUNK)
    zeros_deg = jnp.zeros((ROWS_PER_TILE,), jnp.float32)
    ones_chunk = jnp.ones((DEG_CHUNK,), jnp.float32)
    zeros_rows = jnp.zeros((ROWS_PER_TILE, D_IN), jnp.float32)

    degp = _deg_kernel(dst3d, zeros_deg, ones_chunk)
    degt = degp.T
    u, dinv = _scale(degt, x)
    zp = _agg_kernel(u, src2, dstf, zeros_rows)
    return _dense(zp, u, dinv, w1, b1.reshape(1, H), w2, b2.reshape(1, C))


# speedup vs baseline: 52.3356x; 1.0174x over previous
import functools

import jax
import jax.numpy as jnp
from jax import lax
from jax.experimental import pallas as pl
from jax.experimental.pallas import tpu as pltpu
from jax.experimental.pallas import tpu_sc as plsc

N = 10000
E = 320000
D_IN = 128
H = 256
C = 40

_INFO = plsc.get_sparse_core_info()
NC, NS = _INFO.num_cores, _INFO.num_subcores

NPAD = 10240
ROWS_PER_TILE = NPAD // NS
EDGES_PER_TILE = E // (NC * NS)
CHUNK = 40
NCHUNKS = EDGES_PER_TILE // CHUNK
NBUF = 5
NGROUPS = NCHUNKS // NBUF
DEG_CHUNK = 80
DEG_NCHUNKS = EDGES_PER_TILE // DEG_CHUNK
DEG_WIN = 8

_MESH = plsc.VectorSubcoreMesh(core_axis_name="c", subcore_axis_name="s")


@functools.partial(
    pl.kernel,
    out_type=jax.ShapeDtypeStruct((NC, NPAD), jnp.float32),
    mesh=_MESH,
    scratch_types=[
        [pltpu.VMEM((DEG_CHUNK,), jnp.int32) for _ in range(NBUF)],
        pltpu.VMEM((DEG_CHUNK,), jnp.float32),
        pltpu.VMEM_SHARED((NPAD,), jnp.float32),
        pltpu.SemaphoreType.DMA((NBUF,)),
        pltpu.SemaphoreType.DMA((NBUF,)),
    ],
)
def _deg_kernel(eif_hbm, zeros_hbm, ones_hbm, deg_out, dbufs, ones_v,
                deg_sh, isem, ssem):
    cid = lax.axis_index("c")
    sid = lax.axis_index("s")
    r0 = sid * ROWS_PER_TILE
    tid = cid * NS + sid
    ebase = E + tid * EDGES_PER_TILE
    pltpu.sync_copy(zeros_hbm, deg_sh.at[pl.ds(r0, ROWS_PER_TILE)])
    pltpu.sync_copy(ones_hbm, ones_v)
    plsc.subcore_barrier()

    def fire(i, b):
        pltpu.async_copy(eif_hbm.at[pl.ds(ebase + i * DEG_CHUNK, DEG_CHUNK)],
                         dbufs[b], isem.at[b])

    def visit(i, b, fire_next):
        pltpu.make_async_copy(eif_hbm.at[pl.ds(ebase, DEG_CHUNK)],
                              dbufs[b], isem.at[b]).wait()
        pltpu.async_copy(ones_v, deg_sh.at[dbufs[b]], ssem.at[b], add=True)
        if fire_next:
            pltpu.make_async_copy(ones_v, deg_sh.at[dbufs[b]],
                                  ssem.at[b]).wait()
            fire(i + NBUF, b)

    for b in range(NBUF):
        fire(b, b)

    @pl.loop(0, DEG_NCHUNKS // NBUF - 1)
    def _(g):
        i0 = g * NBUF
        for b in range(NBUF):
            visit(i0 + b, b, True)

    last = (DEG_NCHUNKS // NBUF - 1) * NBUF
    for b in range(NBUF):
        visit(last + b, b, False)
    for b in range(NBUF):
        pltpu.make_async_copy(ones_v, deg_sh.at[dbufs[b]],
                              ssem.at[b]).wait()

    plsc.subcore_barrier()
    pltpu.sync_copy(deg_sh.at[pl.ds(r0, ROWS_PER_TILE)],
                    deg_out.at[cid, pl.ds(r0, ROWS_PER_TILE)])


@functools.partial(
    pl.kernel,
    out_type=jax.ShapeDtypeStruct((NC, NPAD, D_IN), jnp.float32),
    mesh=_MESH,
    scratch_types=[
        pltpu.VMEM((EDGES_PER_TILE,), jnp.int32),
        [pltpu.VMEM((CHUNK,), jnp.int32) for _ in range(NBUF)],
        pltpu.VMEM((NBUF, CHUNK, D_IN), jnp.float32),
        pltpu.VMEM_SHARED((NPAD, D_IN), jnp.float32),
        pltpu.SemaphoreType.DMA((NBUF,)),
        pltpu.SemaphoreType.DMA((NBUF,)),
        pltpu.SemaphoreType.DMA((NBUF,)),
    ],
)
def _agg_kernel(u_hbm, eif_hbm, zrows_hbm, z_out,
                sidx, dbufs, rows, z_sh, gsem, ssem, isem):
    cid = lax.axis_index("c")
    sid = lax.axis_index("s")
    r0 = sid * ROWS_PER_TILE
    tid = cid * NS + sid
    sbase = tid * EDGES_PER_TILE
    dbase = E + tid * EDGES_PER_TILE
    pltpu.sync_copy(zrows_hbm, z_sh.at[pl.ds(r0, ROWS_PER_TILE)])
    pltpu.sync_copy(eif_hbm.at[pl.ds(sbase, EDGES_PER_TILE)], sidx)
    plsc.subcore_barrier()

    def fire(i, b):
        pltpu.async_copy(u_hbm.at[sidx.at[pl.ds(i * CHUNK, CHUNK)]],
                         rows.at[b], gsem.at[b])
        pltpu.async_copy(eif_hbm.at[pl.ds(dbase + i * CHUNK, CHUNK)],
                         dbufs[b], isem.at[b])

    def visit(i, b, fire_next):
        pltpu.make_async_copy(eif_hbm.at[pl.ds(dbase, CHUNK)],
                              dbufs[b], isem.at[b]).wait()
        pltpu.make_async_copy(u_hbm.at[sidx.at[pl.ds(0, CHUNK)]],
                              rows.at[b], gsem.at[b]).wait()
        pltpu.async_copy(rows.at[b], z_sh.at[dbufs[b]], ssem.at[b],
                         add=True)
        if fire_next:
            pltpu.make_async_copy(rows.at[b], z_sh.at[dbufs[b]],
                                  ssem.at[b]).wait()
            fire(i + NBUF, b)

    for b in range(NBUF):
        fire(b, b)

    @pl.loop(0, NGROUPS - 1)
    def _(g):
        i0 = g * NBUF
        for b in range(NBUF):
            visit(i0 + b, b, True)

    last = (NGROUPS - 1) * NBUF
    for b in range(NBUF):
        visit(last + b, b, False)
    for b in range(NBUF):
        pltpu.make_async_copy(rows.at[b], z_sh.at[dbufs[b]],
                              ssem.at[b]).wait()

    plsc.subcore_barrier()
    pltpu.sync_copy(z_sh.at[pl.ds(r0, ROWS_PER_TILE)],
                    z_out.at[cid, pl.ds(r0, ROWS_PER_TILE)])


_ROWBLK = 1000


def _scale_body(degt_ref, x_ref, u_ref, dinv_ref):
    deg = degt_ref[:, 0:1] + degt_ref[:, 1:2] + 1.0
    dinv = lax.rsqrt(deg)
    dinv_ref[...] = dinv
    u_ref[...] = x_ref[...] * dinv


def _scale(degt, x):
    grid = (N // _ROWBLK,)
    return pl.pallas_call(
        _scale_body,
        grid=grid,
        in_specs=[
            pl.BlockSpec((_ROWBLK, NC), lambda i: (i, 0)),
            pl.BlockSpec((_ROWBLK, D_IN), lambda i: (i, 0)),
        ],
        out_specs=[
            pl.BlockSpec((_ROWBLK, D_IN), lambda i: (i, 0)),
            pl.BlockSpec((_ROWBLK, 1), lambda i: (i, 0)),
        ],
        out_shape=[
            jax.ShapeDtypeStruct((N, D_IN), jnp.float32),
            jax.ShapeDtypeStruct((N, 1), jnp.float32),
        ],
    )(degt, x)


def _dense_body(z_ref, u_ref, dinv_ref, w1_ref, b1_ref, w2_ref, b2_ref,
                out_ref):
    zsum = z_ref[0] + z_ref[1] + u_ref[...]
    y = zsum * dinv_ref[...]
    h = jnp.dot(y, w1_ref[...], preferred_element_type=jnp.float32)
    h = jnp.maximum(h + b1_ref[...], 0.0)
    o = jnp.dot(h, w2_ref[...], preferred_element_type=jnp.float32)
    o = o + b2_ref[...]
    m = jnp.max(o, axis=1, keepdims=True)
    lse = jnp.log(jnp.sum(jnp.exp(o - m), axis=1, keepdims=True)) + m
    out_ref[...] = o - lse


def _dense(zp, u, dinv, w1, b1, w2, b2):
    grid = (N // _ROWBLK,)
    return pl.pallas_call(
        _dense_body,
        grid=grid,
        in_specs=[
            pl.BlockSpec((NC, _ROWBLK, D_IN), lambda i: (0, i, 0)),
            pl.BlockSpec((_ROWBLK, D_IN), lambda i: (i, 0)),
            pl.BlockSpec((_ROWBLK, 1), lambda i: (i, 0)),
            pl.BlockSpec((D_IN, H), lambda i: (0, 0)),
            pl.BlockSpec((1, H), lambda i: (0, 0)),
            pl.BlockSpec((H, C), lambda i: (0, 0)),
            pl.BlockSpec((1, C), lambda i: (0, 0)),
        ],
        out_specs=pl.BlockSpec((_ROWBLK, C), lambda i: (i, 0)),
        out_shape=jax.ShapeDtypeStruct((N, C), jnp.float32),
    )(zp, u, dinv, w1, b1, w2, b2)


def kernel(x, edge_index, w1, b1, w2, b2):
    eif = edge_index.reshape(2 * E)
    zeros_deg = jnp.zeros((ROWS_PER_TILE,), jnp.float32)
    ones_chunk = jnp.ones((DEG_CHUNK,), jnp.float32)
    zeros_rows = jnp.zeros((ROWS_PER_TILE, D_IN), jnp.float32)

    degp = _deg_kernel(eif, zeros_deg, ones_chunk)
    degt = degp.T
    u, dinv = _scale(degt, x)
    zp = _agg_kernel(u, eif, zeros_rows)
    return _dense(zp, u, dinv, w1, b1.reshape(1, H), w2, b2.reshape(1, C))
